# baseline (speedup 1.0000x reference)
import functools

import jax
import jax.numpy as jnp
from jax import lax
from jax.experimental import pallas as pl
from jax.experimental.pallas import tpu as pltpu
from jax.experimental.pallas import tpu_sc as plsc

_B, _T, _C = 16, 576, 512
_K, _D = 8192, 256
_N = _B * _T

_M = 512
_NT = _N // _M
_KT = 1024
_NKT = _K // _KT

_NW = 32
_BPW = _N // _NW
_CHUNK = 96
_NCH = _BPW // _CHUNK


def _argmin_body(z2_ref, rn_ref, cbhi_ref, cblo_ref, cbn_ref, out_ref):
    z2b = z2_ref[...]
    rn = rn_ref[...]
    best_val = jnp.full((_M, 1), jnp.inf, jnp.float32)
    best_idx = jnp.zeros((_M, 1), jnp.int32)
    for j in range(_NKT):
        s = (jnp.dot(z2b, cbhi_ref[:, j * _KT:(j + 1) * _KT],
                     preferred_element_type=jnp.float32)
             + jnp.dot(z2b, cblo_ref[:, j * _KT:(j + 1) * _KT],
                       preferred_element_type=jnp.float32))
        d2 = (rn - s) + cbn_ref[:, j * _KT:(j + 1) * _KT]
        tmin = jnp.min(d2, axis=1, keepdims=True)
        ii = lax.broadcasted_iota(jnp.int32, (_M, _KT), 1)
        cand = jnp.where(d2 == tmin, ii, _KT)
        targ = jnp.min(cand, axis=1, keepdims=True) + j * _KT
        upd = tmin < best_val
        best_val = jnp.where(upd, tmin, best_val)
        best_idx = jnp.where(upd, targ, best_idx)
    out_ref[...] = best_idx


def _decode_table_body(cb_ref, w_ref, b_ref, out_ref):
    out_ref[...] = (jnp.dot(cb_ref[...].astype(jnp.bfloat16),
                            w_ref[...].astype(jnp.bfloat16),
                            preferred_element_type=jnp.float32) + b_ref[...])


def _make_sc_gather():
    mesh = plsc.VectorSubcoreMesh(core_axis_name="c", subcore_axis_name="s")

    @functools.partial(
        pl.kernel, mesh=mesh,
        out_type=[jax.ShapeDtypeStruct((_N, _D), jnp.float32),
                  jax.ShapeDtypeStruct((_N, _C), jnp.float32)],
        scratch_types=[
            pltpu.VMEM((_CHUNK,), jnp.int32),
            pltpu.VMEM((_CHUNK, _D), jnp.float32),
            pltpu.VMEM((_CHUNK, _C), jnp.float32),
            pltpu.SemaphoreType.DMA,
            pltpu.SemaphoreType.DMA,
        ],
    )
    def sc_gather(cb_hbm, cw_hbm, idx_hbm, zq_out, xr_out,
                  idx_v, zq_v, cw_v, sem1, sem2):
        wid = lax.axis_index("s") * 2 + lax.axis_index("c")
        base = wid * _BPW
        for c in range(_NCH):
            off = base + c * _CHUNK
            pltpu.sync_copy(idx_hbm.at[pl.ds(off, _CHUNK)], idx_v)
            g1 = pltpu.async_copy(cb_hbm.at[idx_v], zq_v, sem1)
            g2 = pltpu.async_copy(cw_hbm.at[idx_v], cw_v, sem2)
            g1.wait()
            g2.wait()
            pltpu.sync_copy(zq_v, zq_out.at[pl.ds(off, _CHUNK)])
            pltpu.sync_copy(cw_v, xr_out.at[pl.ds(off, _CHUNK)])

    return sc_gather


def kernel(input, W_enc, b_enc, codebook, W_dec, b_dec):
    z_e = jnp.einsum('btc,cd->btd', input, W_enc) + b_enc
    flat = z_e.reshape(-1, _D)
    rn = jnp.sum(flat * flat, axis=1, keepdims=True)
    z2b = (2.0 * flat).astype(jnp.bfloat16)
    cbT = codebook.T
    cb_hi = cbT.astype(jnp.bfloat16)
    cb_lo = (cbT - cb_hi.astype(jnp.float32)).astype(jnp.bfloat16)
    cbn = jnp.sum(codebook * codebook, axis=1)[None, :]
    b_dec2 = b_dec[None, :]

    ind_t = pl.pallas_call(
        _argmin_body,
        grid=(_NT,),
        in_specs=[
            pl.BlockSpec((_M, _D), lambda t: (t, 0)),
            pl.BlockSpec((_M, 1), lambda t: (t, 0)),
            pl.BlockSpec((_D, _K), lambda t: (0, 0)),
            pl.BlockSpec((_D, _K), lambda t: (0, 0)),
            pl.BlockSpec((1, _K), lambda t: (0, 0)),
        ],
        out_specs=pl.BlockSpec((_M, 1), lambda t: (t, 0)),
        out_shape=jax.ShapeDtypeStruct((_N, 1), jnp.int32),
    )(z2b, rn, cb_hi, cb_lo, cbn)
    z_ind = ind_t.reshape(_N)

    cw = pl.pallas_call(
        _decode_table_body,
        grid=(8,),
        in_specs=[
            pl.BlockSpec((_K // 8, _D), lambda i: (i, 0)),
            pl.BlockSpec((_D, _C), lambda i: (0, 0)),
            pl.BlockSpec((1, _C), lambda i: (0, 0)),
        ],
        out_specs=pl.BlockSpec((_K // 8, _C), lambda i: (i, 0)),
        out_shape=jax.ShapeDtypeStruct((_K, _C), jnp.float32),
    )(codebook, W_dec, b_dec2)

    z_q_flat, x_rec_flat = _make_sc_gather()(codebook, cw, z_ind)

    return (x_rec_flat.reshape(_B, _T, _C),
            z_q_flat.reshape(_B, _T, _D),
            z_ind.reshape(_B, _T))


# speedup vs baseline: 1.1641x; 1.1641x over previous
import functools

import jax
import jax.numpy as jnp
from jax import lax
from jax.experimental import pallas as pl
from jax.experimental.pallas import tpu as pltpu
from jax.experimental.pallas import tpu_sc as plsc

_B, _T, _C = 16, 576, 512
_K, _D = 8192, 256
_N = _B * _T

_M = 512
_NT = _N // _M
_KT = 1024
_NKT = _K // _KT

_NW = 32
_BPW = _N // _NW


def _argmin_body(z2_ref, rn_ref, cbt_ref, cbn_ref, out_ref):
    z2b = z2_ref[...]
    rn = rn_ref[...]
    best_val = jnp.full((_M, 1), jnp.inf, jnp.float32)
    best_idx = jnp.zeros((_M, 1), jnp.int32)
    for j in range(_NKT):
        s = jnp.dot(z2b, cbt_ref[:, j * _KT:(j + 1) * _KT],
                    preferred_element_type=jnp.float32)
        d2 = (rn - s) + cbn_ref[:, j * _KT:(j + 1) * _KT]
        tmin = jnp.min(d2, axis=1, keepdims=True)
        ii = lax.broadcasted_iota(jnp.int32, (_M, _KT), 1)
        cand = jnp.where(d2 == tmin, ii, _KT)
        targ = jnp.min(cand, axis=1, keepdims=True) + j * _KT
        upd = tmin < best_val
        best_val = jnp.where(upd, tmin, best_val)
        best_idx = jnp.where(upd, targ, best_idx)
    out_ref[...] = best_idx


def _decode_body(zq_ref, w_ref, b_ref, out_ref):
    out_ref[...] = (jnp.dot(zq_ref[...].astype(jnp.bfloat16),
                            w_ref[...].astype(jnp.bfloat16),
                            preferred_element_type=jnp.float32) + b_ref[...])


def _make_sc_gather():
    mesh = plsc.VectorSubcoreMesh(core_axis_name="c", subcore_axis_name="s")

    @functools.partial(
        pl.kernel, mesh=mesh,
        out_type=jax.ShapeDtypeStruct((_N, _D), jnp.float32),
        scratch_types=[
            pltpu.VMEM((_BPW,), jnp.int32),
            pltpu.VMEM((_BPW, _D), jnp.float32),
            pltpu.SemaphoreType.DMA,
        ],
    )
    def sc_gather(cb_hbm, idx_hbm, zq_out, idx_v, zq_v, sem):
        wid = lax.axis_index("s") * 2 + lax.axis_index("c")
        base = wid * _BPW
        pltpu.sync_copy(idx_hbm.at[pl.ds(base, _BPW)], idx_v)
        pltpu.async_copy(cb_hbm.at[idx_v], zq_v, sem).wait()
        pltpu.sync_copy(zq_v, zq_out.at[pl.ds(base, _BPW)])

    return sc_gather


def kernel(input, W_enc, b_enc, codebook, W_dec, b_dec):
    z_e = jnp.einsum('btc,cd->btd', input, W_enc) + b_enc
    flat = z_e.reshape(-1, _D)
    rn = jnp.sum(flat * flat, axis=1, keepdims=True)
    z2b = (2.0 * flat).astype(jnp.bfloat16)
    cbT = codebook.T.astype(jnp.bfloat16)
    cbn = jnp.sum(codebook * codebook, axis=1)[None, :]
    b_dec2 = b_dec[None, :]

    ind_t = pl.pallas_call(
        _argmin_body,
        grid=(_NT,),
        in_specs=[
            pl.BlockSpec((_M, _D), lambda t: (t, 0)),
            pl.BlockSpec((_M, 1), lambda t: (t, 0)),
            pl.BlockSpec((_D, _K), lambda t: (0, 0)),
            pl.BlockSpec((1, _K), lambda t: (0, 0)),
        ],
        out_specs=pl.BlockSpec((_M, 1), lambda t: (t, 0)),
        out_shape=jax.ShapeDtypeStruct((_N, 1), jnp.int32),
    )(z2b, rn, cbT, cbn)
    z_ind = ind_t.reshape(_N)

    z_q_flat = _make_sc_gather()(codebook, z_ind)

    x_rec_flat = pl.pallas_call(
        _decode_body,
        grid=(_NT,),
        in_specs=[
            pl.BlockSpec((_M, _D), lambda t: (t, 0)),
            pl.BlockSpec((_D, _C), lambda t: (0, 0)),
            pl.BlockSpec((1, _C), lambda t: (0, 0)),
        ],
        out_specs=pl.BlockSpec((_M, _C), lambda t: (t, 0)),
        out_shape=jax.ShapeDtypeStruct((_N, _C), jnp.float32),
    )(z_q_flat, W_dec, b_dec2)

    return (x_rec_flat.reshape(_B, _T, _C),
            z_q_flat.reshape(_B, _T, _D),
            z_ind.reshape(_B, _T))


# speedup vs baseline: 1.2224x; 1.0501x over previous
import functools

import jax
import jax.numpy as jnp
from jax import lax
from jax.experimental import pallas as pl
from jax.experimental.pallas import tpu as pltpu
from jax.experimental.pallas import tpu_sc as plsc

_B, _T, _C = 16, 576, 512
_K, _D = 8192, 256
_N = _B * _T

_M = 1024
_NT = _N // _M
_KT = 1024
_NKT = _K // _KT

_NW = 32
_BPW = _N // _NW


def _argmin_body(z2_ref, rn_ref, cbt_ref, cbn_ref, out_ref):
    z2b = z2_ref[...]
    rn = rn_ref[...]
    best_val = jnp.full((_M, 1), jnp.inf, jnp.float32)
    best_idx = jnp.zeros((_M, 1), jnp.int32)
    for j in range(_NKT):
        s = jnp.dot(z2b, cbt_ref[:, j * _KT:(j + 1) * _KT],
                    preferred_element_type=jnp.float32)
        d2 = (rn - s) + cbn_ref[:, j * _KT:(j + 1) * _KT]
        tmin = jnp.min(d2, axis=1, keepdims=True)
        ii = lax.broadcasted_iota(jnp.int32, (_M, _KT), 1)
        cand = jnp.where(d2 == tmin, ii, _KT)
        targ = jnp.min(cand, axis=1, keepdims=True) + j * _KT
        upd = tmin < best_val
        best_val = jnp.where(upd, tmin, best_val)
        best_idx = jnp.where(upd, targ, best_idx)
    out_ref[...] = best_idx


def _decode_body(zq_ref, w_ref, b_ref, out_ref):
    out_ref[...] = (jnp.dot(zq_ref[...].astype(jnp.bfloat16),
                            w_ref[...].astype(jnp.bfloat16),
                            preferred_element_type=jnp.float32) + b_ref[...])


def _make_sc_gather():
    mesh = plsc.VectorSubcoreMesh(core_axis_name="c", subcore_axis_name="s")

    @functools.partial(
        pl.kernel, mesh=mesh,
        out_type=jax.ShapeDtypeStruct((_N, _D), jnp.float32),
        scratch_types=[
            pltpu.VMEM((_BPW,), jnp.int32),
            pltpu.VMEM((_BPW, _D), jnp.float32),
            pltpu.SemaphoreType.DMA,
        ],
    )
    def sc_gather(cb_hbm, idx_hbm, zq_out, idx_v, zq_v, sem):
        wid = lax.axis_index("s") * 2 + lax.axis_index("c")
        base = wid * _BPW
        pltpu.sync_copy(idx_hbm.at[pl.ds(base, _BPW)], idx_v)
        pltpu.async_copy(cb_hbm.at[idx_v], zq_v, sem).wait()
        pltpu.sync_copy(zq_v, zq_out.at[pl.ds(base, _BPW)])

    return sc_gather


def kernel(input, W_enc, b_enc, codebook, W_dec, b_dec):
    z_e = jnp.einsum('btc,cd->btd', input, W_enc) + b_enc
    flat = z_e.reshape(-1, _D)
    rn = jnp.sum(flat * flat, axis=1, keepdims=True)
    z2b = (2.0 * flat).astype(jnp.bfloat16)
    cbT = codebook.T.astype(jnp.bfloat16)
    cbn = jnp.sum(codebook * codebook, axis=1)[None, :]
    b_dec2 = b_dec[None, :]

    ind_t = pl.pallas_call(
        _argmin_body,
        grid=(_NT,),
        in_specs=[
            pl.BlockSpec((_M, _D), lambda t: (t, 0)),
            pl.BlockSpec((_M, 1), lambda t: (t, 0)),
            pl.BlockSpec((_D, _K), lambda t: (0, 0)),
            pl.BlockSpec((1, _K), lambda t: (0, 0)),
        ],
        out_specs=pl.BlockSpec((_M, 1), lambda t: (t, 0)),
        out_shape=jax.ShapeDtypeStruct((_N, 1), jnp.int32),
    )(z2b, rn, cbT, cbn)
    z_ind = ind_t.reshape(_N)

    z_q_flat = _make_sc_gather()(codebook, z_ind)

    x_rec_flat = pl.pallas_call(
        _decode_body,
        grid=(_NT,),
        in_specs=[
            pl.BlockSpec((_M, _D), lambda t: (t, 0)),
            pl.BlockSpec((_D, _C), lambda t: (0, 0)),
            pl.BlockSpec((1, _C), lambda t: (0, 0)),
        ],
        out_specs=pl.BlockSpec((_M, _C), lambda t: (t, 0)),
        out_shape=jax.ShapeDtypeStruct((_N, _C), jnp.float32),
    )(z_q_flat, W_dec, b_dec2)

    return (x_rec_flat.reshape(_B, _T, _C),
            z_q_flat.reshape(_B, _T, _D),
            z_ind.reshape(_B, _T))


# speedup vs baseline: 1.2791x; 1.0464x over previous
import functools

import jax
import jax.numpy as jnp
from jax import lax
from jax.experimental import pallas as pl
from jax.experimental.pallas import tpu as pltpu
from jax.experimental.pallas import tpu_sc as plsc

_B, _T, _C = 16, 576, 512
_K, _D = 8192, 256
_N = _B * _T

_M = 1024
_NT = _N // _M
_KT = 2048
_NKT = _K // _KT

_NW = 32
_BPW = _N // _NW


def _argmin_body(z2_ref, rn_ref, cbt_ref, cbn_ref, out_ref):
    z2b = z2_ref[...]
    rn = rn_ref[...]
    best_val = jnp.full((_M, 1), jnp.inf, jnp.float32)
    best_idx = jnp.zeros((_M, 1), jnp.int32)
    for j in range(_NKT):
        s = jnp.dot(z2b, cbt_ref[:, j * _KT:(j + 1) * _KT],
                    preferred_element_type=jnp.float32)
        d2 = (rn - s) + cbn_ref[:, j * _KT:(j + 1) * _KT]
        tmin = jnp.min(d2, axis=1, keepdims=True)
        ii = lax.broadcasted_iota(jnp.int32, (_M, _KT), 1)
        cand = jnp.where(d2 == tmin, ii, _KT)
        targ = jnp.min(cand, axis=1, keepdims=True) + j * _KT
        upd = tmin < best_val
        best_val = jnp.where(upd, tmin, best_val)
        best_idx = jnp.where(upd, targ, best_idx)
    out_ref[...] = best_idx


def _decode_body(zq_ref, w_ref, b_ref, out_ref):
    out_ref[...] = (jnp.dot(zq_ref[...].astype(jnp.bfloat16),
                            w_ref[...].astype(jnp.bfloat16),
                            preferred_element_type=jnp.float32) + b_ref[...])


def _make_sc_gather():
    mesh = plsc.VectorSubcoreMesh(core_axis_name="c", subcore_axis_name="s")

    @functools.partial(
        pl.kernel, mesh=mesh,
        out_type=jax.ShapeDtypeStruct((_N, _D), jnp.float32),
        scratch_types=[
            pltpu.VMEM((_BPW,), jnp.int32),
            pltpu.VMEM((_BPW, _D), jnp.float32),
            pltpu.SemaphoreType.DMA,
        ],
    )
    def sc_gather(cb_hbm, idx_hbm, zq_out, idx_v, zq_v, sem):
        wid = lax.axis_index("s") * 2 + lax.axis_index("c")
        base = wid * _BPW
        pltpu.sync_copy(idx_hbm.at[pl.ds(base, _BPW)], idx_v)
        pltpu.async_copy(cb_hbm.at[idx_v], zq_v, sem).wait()
        pltpu.sync_copy(zq_v, zq_out.at[pl.ds(base, _BPW)])

    return sc_gather


def kernel(input, W_enc, b_enc, codebook, W_dec, b_dec):
    z_e = jnp.einsum('btc,cd->btd', input, W_enc) + b_enc
    flat = z_e.reshape(-1, _D)
    rn = jnp.sum(flat * flat, axis=1, keepdims=True)
    z2b = (2.0 * flat).astype(jnp.bfloat16)
    cbT = codebook.T.astype(jnp.bfloat16)
    cbn = jnp.sum(codebook * codebook, axis=1)[None, :]
    b_dec2 = b_dec[None, :]

    ind_t = pl.pallas_call(
        _argmin_body,
        grid=(_NT,),
        in_specs=[
            pl.BlockSpec((_M, _D), lambda t: (t, 0)),
            pl.BlockSpec((_M, 1), lambda t: (t, 0)),
            pl.BlockSpec((_D, _K), lambda t: (0, 0)),
            pl.BlockSpec((1, _K), lambda t: (0, 0)),
        ],
        out_specs=pl.BlockSpec((_M, 1), lambda t: (t, 0)),
        out_shape=jax.ShapeDtypeStruct((_N, 1), jnp.int32),
    )(z2b, rn, cbT, cbn)
    z_ind = ind_t.reshape(_N)

    z_q_flat = _make_sc_gather()(codebook, z_ind)

    x_rec_flat = pl.pallas_call(
        _decode_body,
        grid=(_NT,),
        in_specs=[
            pl.BlockSpec((_M, _D), lambda t: (t, 0)),
            pl.BlockSpec((_D, _C), lambda t: (0, 0)),
            pl.BlockSpec((1, _C), lambda t: (0, 0)),
        ],
        out_specs=pl.BlockSpec((_M, _C), lambda t: (t, 0)),
        out_shape=jax.ShapeDtypeStruct((_N, _C), jnp.float32),
    )(z_q_flat, W_dec, b_dec2)

    return (x_rec_flat.reshape(_B, _T, _C),
            z_q_flat.reshape(_B, _T, _D),
            z_ind.reshape(_B, _T))


# speedup vs baseline: 1.3222x; 1.0337x over previous
import functools

import jax
import jax.numpy as jnp
from jax import lax
from jax.experimental import pallas as pl
from jax.experimental.pallas import tpu as pltpu
from jax.experimental.pallas import tpu_sc as plsc

_B, _T, _C = 16, 576, 512
_K, _D = 8192, 256
_N = _B * _T

_M = 1024
_NT = _N // _M
_KT = 4096
_NKT = _K // _KT

_NW = 32
_BPW = _N // _NW


def _argmin_body(z2_ref, rn_ref, cbt_ref, cbn_ref, out_ref):
    z2b = z2_ref[...]
    rn = rn_ref[...]
    best_val = jnp.full((_M, 1), jnp.inf, jnp.float32)
    best_idx = jnp.zeros((_M, 1), jnp.int32)
    for j in range(_NKT):
        s = jnp.dot(z2b, cbt_ref[:, j * _KT:(j + 1) * _KT],
                    preferred_element_type=jnp.float32)
        d2 = (rn - s) + cbn_ref[:, j * _KT:(j + 1) * _KT]
        tmin = jnp.min(d2, axis=1, keepdims=True)
        ii = lax.broadcasted_iota(jnp.int32, (_M, _KT), 1)
        cand = jnp.where(d2 == tmin, ii, _KT)
        targ = jnp.min(cand, axis=1, keepdims=True) + j * _KT
        upd = tmin < best_val
        best_val = jnp.where(upd, tmin, best_val)
        best_idx = jnp.where(upd, targ, best_idx)
    out_ref[...] = best_idx


def _decode_body(zq_ref, w_ref, b_ref, out_ref):
    out_ref[...] = (jnp.dot(zq_ref[...].astype(jnp.bfloat16),
                            w_ref[...].astype(jnp.bfloat16),
                            preferred_element_type=jnp.float32) + b_ref[...])


def _make_sc_gather():
    mesh = plsc.VectorSubcoreMesh(core_axis_name="c", subcore_axis_name="s")

    @functools.partial(
        pl.kernel, mesh=mesh,
        out_type=jax.ShapeDtypeStruct((_N, _D), jnp.float32),
        scratch_types=[
            pltpu.VMEM((_BPW,), jnp.int32),
            pltpu.VMEM((_BPW, _D), jnp.float32),
            pltpu.SemaphoreType.DMA,
        ],
    )
    def sc_gather(cb_hbm, idx_hbm, zq_out, idx_v, zq_v, sem):
        wid = lax.axis_index("s") * 2 + lax.axis_index("c")
        base = wid * _BPW
        pltpu.sync_copy(idx_hbm.at[pl.ds(base, _BPW)], idx_v)
        pltpu.async_copy(cb_hbm.at[idx_v], zq_v, sem).wait()
        pltpu.sync_copy(zq_v, zq_out.at[pl.ds(base, _BPW)])

    return sc_gather


def kernel(input, W_enc, b_enc, codebook, W_dec, b_dec):
    z_e = jnp.einsum('btc,cd->btd', input, W_enc) + b_enc
    flat = z_e.reshape(-1, _D)
    rn = jnp.sum(flat * flat, axis=1, keepdims=True)
    z2b = (2.0 * flat).astype(jnp.bfloat16)
    cbT = codebook.T.astype(jnp.bfloat16)
    cbn = jnp.sum(codebook * codebook, axis=1)[None, :]
    b_dec2 = b_dec[None, :]

    ind_t = pl.pallas_call(
        _argmin_body,
        grid=(_NT,),
        in_specs=[
            pl.BlockSpec((_M, _D), lambda t: (t, 0)),
            pl.BlockSpec((_M, 1), lambda t: (t, 0)),
            pl.BlockSpec((_D, _K), lambda t: (0, 0)),
            pl.BlockSpec((1, _K), lambda t: (0, 0)),
        ],
        out_specs=pl.BlockSpec((_M, 1), lambda t: (t, 0)),
        out_shape=jax.ShapeDtypeStruct((_N, 1), jnp.int32),
    )(z2b, rn, cbT, cbn)
    z_ind = ind_t.reshape(_N)

    z_q_flat = _make_sc_gather()(codebook, z_ind)

    x_rec_flat = pl.pallas_call(
        _decode_body,
        grid=(_NT,),
        in_specs=[
            pl.BlockSpec((_M, _D), lambda t: (t, 0)),
            pl.BlockSpec((_D, _C), lambda t: (0, 0)),
            pl.BlockSpec((1, _C), lambda t: (0, 0)),
        ],
        out_specs=pl.BlockSpec((_M, _C), lambda t: (t, 0)),
        out_shape=jax.ShapeDtypeStruct((_N, _C), jnp.float32),
    )(z_q_flat, W_dec, b_dec2)

    return (x_rec_flat.reshape(_B, _T, _C),
            z_q_flat.reshape(_B, _T, _D),
            z_ind.reshape(_B, _T))


# speedup vs baseline: 1.3336x; 1.0086x over previous
import functools

import jax
import jax.numpy as jnp
from jax import lax
from jax.experimental import pallas as pl
from jax.experimental.pallas import tpu as pltpu
from jax.experimental.pallas import tpu_sc as plsc

_B, _T, _C = 16, 576, 512
_K, _D = 8192, 256
_N = _B * _T

_M = 1024
_NT = _N // _M
_KT = 8192
_NKT = _K // _KT

_NW = 32
_BPW = _N // _NW


def _argmin_body(z2_ref, rn_ref, cbt_ref, cbn_ref, out_ref):
    z2b = z2_ref[...]
    rn = rn_ref[...]
    best_val = jnp.full((_M, 1), jnp.inf, jnp.float32)
    best_idx = jnp.zeros((_M, 1), jnp.int32)
    for j in range(_NKT):
        s = jnp.dot(z2b, cbt_ref[:, j * _KT:(j + 1) * _KT],
                    preferred_element_type=jnp.float32)
        d2 = (rn - s) + cbn_ref[:, j * _KT:(j + 1) * _KT]
        tmin = jnp.min(d2, axis=1, keepdims=True)
        ii = lax.broadcasted_iota(jnp.int32, (_M, _KT), 1)
        cand = jnp.where(d2 == tmin, ii, _KT)
        targ = jnp.min(cand, axis=1, keepdims=True) + j * _KT
        upd = tmin < best_val
        best_val = jnp.where(upd, tmin, best_val)
        best_idx = jnp.where(upd, targ, best_idx)
    out_ref[...] = best_idx


def _decode_body(zq_ref, w_ref, b_ref, out_ref):
    out_ref[...] = (jnp.dot(zq_ref[...].astype(jnp.bfloat16),
                            w_ref[...].astype(jnp.bfloat16),
                            preferred_element_type=jnp.float32) + b_ref[...])


def _make_sc_gather():
    mesh = plsc.VectorSubcoreMesh(core_axis_name="c", subcore_axis_name="s")

    @functools.partial(
        pl.kernel, mesh=mesh,
        out_type=jax.ShapeDtypeStruct((_N, _D), jnp.float32),
        scratch_types=[
            pltpu.VMEM((_BPW,), jnp.int32),
            pltpu.VMEM((_BPW, _D), jnp.float32),
            pltpu.SemaphoreType.DMA,
        ],
    )
    def sc_gather(cb_hbm, idx_hbm, zq_out, idx_v, zq_v, sem):
        wid = lax.axis_index("s") * 2 + lax.axis_index("c")
        base = wid * _BPW
        pltpu.sync_copy(idx_hbm.at[pl.ds(base, _BPW)], idx_v)
        pltpu.async_copy(cb_hbm.at[idx_v], zq_v, sem).wait()
        pltpu.sync_copy(zq_v, zq_out.at[pl.ds(base, _BPW)])

    return sc_gather


def kernel(input, W_enc, b_enc, codebook, W_dec, b_dec):
    z_e = jnp.einsum('btc,cd->btd', input, W_enc) + b_enc
    flat = z_e.reshape(-1, _D)
    rn = jnp.sum(flat * flat, axis=1, keepdims=True)
    z2b = (2.0 * flat).astype(jnp.bfloat16)
    cbT = codebook.T.astype(jnp.bfloat16)
    cbn = jnp.sum(codebook * codebook, axis=1)[None, :]
    b_dec2 = b_dec[None, :]

    ind_t = pl.pallas_call(
        _argmin_body,
        grid=(_NT,),
        in_specs=[
            pl.BlockSpec((_M, _D), lambda t: (t, 0)),
            pl.BlockSpec((_M, 1), lambda t: (t, 0)),
            pl.BlockSpec((_D, _K), lambda t: (0, 0)),
            pl.BlockSpec((1, _K), lambda t: (0, 0)),
        ],
        out_specs=pl.BlockSpec((_M, 1), lambda t: (t, 0)),
        out_shape=jax.ShapeDtypeStruct((_N, 1), jnp.int32),
    )(z2b, rn, cbT, cbn)
    z_ind = ind_t.reshape(_N)

    z_q_flat = _make_sc_gather()(codebook, z_ind)

    x_rec_flat = pl.pallas_call(
        _decode_body,
        grid=(_NT,),
        in_specs=[
            pl.BlockSpec((_M, _D), lambda t: (t, 0)),
            pl.BlockSpec((_D, _C), lambda t: (0, 0)),
            pl.BlockSpec((1, _C), lambda t: (0, 0)),
        ],
        out_specs=pl.BlockSpec((_M, _C), lambda t: (t, 0)),
        out_shape=jax.ShapeDtypeStruct((_N, _C), jnp.float32),
    )(z_q_flat, W_dec, b_dec2)

    return (x_rec_flat.reshape(_B, _T, _C),
            z_q_flat.reshape(_B, _T, _D),
            z_ind.reshape(_B, _T))
